# baseline (speedup 1.0000x reference)
import functools

import jax
import jax.numpy as jnp
from jax import lax
from jax.experimental import pallas as pl
from jax.experimental.pallas import tpu as pltpu
from jax.experimental.pallas import tpu_sc as plsc

D = 64
LANES = 16


@functools.lru_cache(maxsize=None)
def _make_gather_sum(N, L, C):
    NC, NS = 2, 16
    NW = NC * NS
    assert N % (NW * C) == 0
    rows_per_w = N // NW
    steps = rows_per_w // C
    assert steps % 2 == 0
    CL = C * L
    mesh = plsc.VectorSubcoreMesh(core_axis_name="c", subcore_axis_name="s")

    @functools.partial(
        pl.kernel,
        out_type=jax.ShapeDtypeStruct((N, D), jnp.float32),
        mesh=mesh,
        scratch_types=[
            pltpu.VMEM((CL,), jnp.int32),
            pltpu.VMEM((CL,), jnp.int32),
            pltpu.VMEM((CL + LANES,), jnp.int32),
            pltpu.VMEM((CL + LANES,), jnp.int32),
            pltpu.VMEM((CL, 2 * D), jnp.float32),
            pltpu.VMEM((CL, 2 * D), jnp.float32),
            pltpu.VMEM((C, D), jnp.float32),
            pltpu.VMEM((C, D), jnp.float32),
            pltpu.SemaphoreType.DMA,
            pltpu.SemaphoreType.DMA,
            pltpu.SemaphoreType.DMA,
            pltpu.SemaphoreType.DMA,
        ],
    )
    def gather_sum(idx2_hbm, offs_hbm, emb2_hbm, out_hbm,
                   idx0, idx1, off0, off1, rows0, rows1, out0, out1,
                   isem0, isem1, gsem0, gsem1):
        wid = lax.axis_index("s") * NC + lax.axis_index("c")
        row0_w = wid * rows_per_w
        tok0_w = row0_w * L

        def stage_idx(chunk, idxv, sem):
            pltpu.async_copy(
                idx2_hbm.at[pl.ds(tok0_w + chunk * CL, CL)], idxv, sem)

        def stage_off(chunk, offv, sem):
            pltpu.async_copy(
                offs_hbm.at[pl.ds(tok0_w + chunk * CL, CL)],
                offv.at[pl.ds(0, CL)], sem)

        def wait_stage(idxv, sem):
            pltpu.make_async_copy(
                idx2_hbm.at[pl.ds(tok0_w, CL)], idxv, sem).wait()

        def gather(idxv, rows, sem):
            pltpu.async_copy(emb2_hbm.at[idxv], rows, sem)

        def wait_gather(idxv, rows, sem):
            pltpu.make_async_copy(emb2_hbm.at[idxv], rows, sem).wait()

        def reduce_chunk(chunk, rows, offv, outv):
            def reduce_row(r, carry2):
                base = r * L

                def off_at(g):
                    return offv[pl.ds(g, LANES)][0]

                o0 = off_at(base)
                accs = [rows[base, pl.ds(o0 + k * LANES, LANES)]
                        for k in range(D // LANES)]
                for j in range(1, L):
                    oj = off_at(base + j)
                    accs = [a + rows[base + j, pl.ds(oj + k * LANES, LANES)]
                            for k, a in enumerate(accs)]
                for k, a in enumerate(accs):
                    outv[r, pl.ds(k * LANES, LANES)] = a
                return carry2

            lax.fori_loop(0, C, reduce_row, 0)
            pltpu.sync_copy(outv, out_hbm.at[pl.ds(row0_w + chunk * C, C)])

        pltpu.sync_copy(idx2_hbm.at[pl.ds(tok0_w, CL)], idx0)
        stage_off(0, off0, isem0)
        gather(idx0, rows0, gsem0)
        stage_idx(1, idx1, isem1)
        stage_off(1, off1, isem1)

        def body(i, carry):
            c0 = 2 * i
            n2 = jnp.minimum(c0 + 2, steps - 1)
            n3 = jnp.minimum(c0 + 3, steps - 1)
            wait_stage(idx1, isem1)
            wait_stage(idx1, isem1)
            gather(idx1, rows1, gsem1)
            wait_stage(idx0, isem0)
            wait_gather(idx0, rows0, gsem0)
            reduce_chunk(c0, rows0, off0, out0)
            stage_idx(n2, idx0, isem0)
            wait_stage(idx0, isem0)
            gather(idx0, rows0, gsem0)
            stage_off(n2, off0, isem0)
            wait_gather(idx1, rows1, gsem1)
            reduce_chunk(c0 + 1, rows1, off1, out1)
            stage_idx(n3, idx1, isem1)
            stage_off(n3, off1, isem1)
            return carry

        lax.fori_loop(0, steps // 2, body, 0)
        wait_stage(idx0, isem0)
        wait_gather(idx0, rows0, gsem0)
        wait_stage(idx1, isem1)
        wait_stage(idx1, isem1)

    return gather_sum


def _finish_body(tok_ref, sums_ref, w_ref, b_ref, out_ref):
    tok = tok_ref[...]
    cnt = jnp.sum((tok != 0).astype(jnp.float32), axis=1, keepdims=True)
    inv = 1.0 / jnp.maximum(cnt, 1.0)
    mean = sums_ref[...] * inv
    acc = lax.dot_general(mean, w_ref[...], (((1,), (1,)), ((), ())),
                          preferred_element_type=jnp.float32)
    out_ref[...] = jnp.maximum(acc + b_ref[...], 0.0)


def _finish(tok, sums, W, b, block_rows=2048):
    N, L = tok.shape
    assert N % block_rows == 0
    return pl.pallas_call(
        _finish_body,
        grid=(N // block_rows,),
        in_specs=[
            pl.BlockSpec((block_rows, L), lambda i: (i, 0)),
            pl.BlockSpec((block_rows, D), lambda i: (i, 0)),
            pl.BlockSpec((D, D), lambda i: (0, 0)),
            pl.BlockSpec((1, D), lambda i: (0, 0)),
        ],
        out_specs=pl.BlockSpec((block_rows, D), lambda i: (i, 0)),
        out_shape=jax.ShapeDtypeStruct((N, D), jnp.float32),
    )(tok, sums, W, b.reshape(1, D))


def kernel(sample, emb, W, b):
    L = sample.shape[-1]
    flat = sample.reshape(-1, L).astype(jnp.int32)
    N = flat.shape[0]
    toks = flat.reshape(-1)
    idx2 = toks >> 1
    offs = (toks & 1) << 6
    emb2 = emb.reshape(emb.shape[0] // 2, 2 * D)
    sums = _make_gather_sum(N, L, 8)(idx2, offs, emb2)
    out = _finish(flat, sums, W, b)
    return out.reshape(sample.shape[:-1] + (D,))


# speedup vs baseline: 1.0002x; 1.0002x over previous
import functools

import jax
import jax.numpy as jnp
from jax import lax
from jax.experimental import pallas as pl
from jax.experimental.pallas import tpu as pltpu
from jax.experimental.pallas import tpu_sc as plsc

D = 64
LANES = 16


@functools.lru_cache(maxsize=None)
def _make_gather_sum(N, L, C):
    NC, NS = 2, 16
    NW = NC * NS
    assert N % (NW * C) == 0
    rows_per_w = N // NW
    steps = rows_per_w // C
    assert steps % 2 == 0
    CL = C * L
    mesh = plsc.VectorSubcoreMesh(core_axis_name="c", subcore_axis_name="s")

    @functools.partial(
        pl.kernel,
        out_type=jax.ShapeDtypeStruct((N, D), jnp.float32),
        mesh=mesh,
        scratch_types=[
            pltpu.VMEM((CL,), jnp.int32),
            pltpu.VMEM((CL,), jnp.int32),
            pltpu.VMEM((CL + LANES,), jnp.int32),
            pltpu.VMEM((CL + LANES,), jnp.int32),
            pltpu.VMEM((CL, 2 * D), jnp.float32),
            pltpu.VMEM((CL, 2 * D), jnp.float32),
            pltpu.VMEM((C, D), jnp.float32),
            pltpu.VMEM((C, D), jnp.float32),
            pltpu.SemaphoreType.DMA,
            pltpu.SemaphoreType.DMA,
            pltpu.SemaphoreType.DMA,
            pltpu.SemaphoreType.DMA,
        ],
        compiler_params=pltpu.CompilerParams(use_tc_tiling_on_sc=True),
    )
    def gather_sum(idx2_hbm, offs_hbm, emb2_hbm, out_hbm,
                   idx0, idx1, off0, off1, rows0, rows1, out0, out1,
                   isem0, isem1, gsem0, gsem1):
        wid = lax.axis_index("s") * NC + lax.axis_index("c")
        row0_w = wid * rows_per_w
        tok0_w = row0_w * L

        def stage_idx(chunk, idxv, sem):
            pltpu.async_copy(
                idx2_hbm.at[pl.ds(tok0_w + chunk * CL, CL)], idxv, sem)

        def stage_off(chunk, offv, sem):
            pltpu.async_copy(
                offs_hbm.at[pl.ds(tok0_w + chunk * CL, CL)],
                offv.at[pl.ds(0, CL)], sem)

        def wait_stage(idxv, sem):
            pltpu.make_async_copy(
                idx2_hbm.at[pl.ds(tok0_w, CL)], idxv, sem).wait()

        def gather(idxv, rows, sem):
            pltpu.async_copy(emb2_hbm.at[idxv], rows, sem)

        def wait_gather(idxv, rows, sem):
            pltpu.make_async_copy(emb2_hbm.at[idxv], rows, sem).wait()

        def reduce_chunk(chunk, rows, offv, outv):
            def reduce_row(r, carry2):
                base = r * L

                def off_at(g):
                    return offv[pl.ds(g, LANES)][0]

                o0 = off_at(base)
                accs = [rows[base, pl.ds(o0 + k * LANES, LANES)]
                        for k in range(D // LANES)]
                for j in range(1, L):
                    oj = off_at(base + j)
                    accs = [a + rows[base + j, pl.ds(oj + k * LANES, LANES)]
                            for k, a in enumerate(accs)]
                for k, a in enumerate(accs):
                    outv[r, pl.ds(k * LANES, LANES)] = a
                return carry2

            lax.fori_loop(0, C, reduce_row, 0)
            pltpu.sync_copy(outv, out_hbm.at[pl.ds(row0_w + chunk * C, C)])

        pltpu.sync_copy(idx2_hbm.at[pl.ds(tok0_w, CL)], idx0)
        stage_off(0, off0, isem0)
        gather(idx0, rows0, gsem0)
        stage_idx(1, idx1, isem1)
        stage_off(1, off1, isem1)

        def body(i, carry):
            c0 = 2 * i
            n2 = jnp.minimum(c0 + 2, steps - 1)
            n3 = jnp.minimum(c0 + 3, steps - 1)
            wait_stage(idx1, isem1)
            wait_stage(idx1, isem1)
            gather(idx1, rows1, gsem1)
            wait_stage(idx0, isem0)
            wait_gather(idx0, rows0, gsem0)
            reduce_chunk(c0, rows0, off0, out0)
            stage_idx(n2, idx0, isem0)
            wait_stage(idx0, isem0)
            gather(idx0, rows0, gsem0)
            stage_off(n2, off0, isem0)
            wait_gather(idx1, rows1, gsem1)
            reduce_chunk(c0 + 1, rows1, off1, out1)
            stage_idx(n3, idx1, isem1)
            stage_off(n3, off1, isem1)
            return carry

        lax.fori_loop(0, steps // 2, body, 0)
        wait_stage(idx0, isem0)
        wait_gather(idx0, rows0, gsem0)
        wait_stage(idx1, isem1)
        wait_stage(idx1, isem1)

    return gather_sum


def _finish_body(tok_ref, sums_ref, w_ref, b_ref, out_ref):
    tok = tok_ref[...]
    cnt = jnp.sum((tok != 0).astype(jnp.float32), axis=1, keepdims=True)
    inv = 1.0 / jnp.maximum(cnt, 1.0)
    mean = sums_ref[...] * inv
    acc = lax.dot_general(mean, w_ref[...], (((1,), (1,)), ((), ())),
                          preferred_element_type=jnp.float32)
    out_ref[...] = jnp.maximum(acc + b_ref[...], 0.0)


def _finish(tok, sums, W, b, block_rows=2048):
    N, L = tok.shape
    assert N % block_rows == 0
    return pl.pallas_call(
        _finish_body,
        grid=(N // block_rows,),
        in_specs=[
            pl.BlockSpec((block_rows, L), lambda i: (i, 0)),
            pl.BlockSpec((block_rows, D), lambda i: (i, 0)),
            pl.BlockSpec((D, D), lambda i: (0, 0)),
            pl.BlockSpec((1, D), lambda i: (0, 0)),
        ],
        out_specs=pl.BlockSpec((block_rows, D), lambda i: (i, 0)),
        out_shape=jax.ShapeDtypeStruct((N, D), jnp.float32),
    )(tok, sums, W, b.reshape(1, D))


def kernel(sample, emb, W, b):
    L = sample.shape[-1]
    flat = sample.reshape(-1, L).astype(jnp.int32)
    N = flat.shape[0]
    toks = flat.reshape(-1)
    idx2 = toks >> 1
    offs = (toks & 1) << 6
    emb2 = emb.reshape(emb.shape[0] // 2, 2 * D)
    sums = _make_gather_sum(N, L, 8)(idx2, offs, emb2)
    out = _finish(flat, sums, W, b)
    return out.reshape(sample.shape[:-1] + (D,))


# speedup vs baseline: 1.1637x; 1.1635x over previous
import functools

import jax
import jax.numpy as jnp
from jax import lax
from jax.experimental import pallas as pl
from jax.experimental.pallas import tpu as pltpu
from jax.experimental.pallas import tpu_sc as plsc

D = 64
LANES = 16


@functools.lru_cache(maxsize=None)
def _make_gather_sum(N, L, C):
    NC, NS = 2, 16
    NW = NC * NS
    assert N % (NW * C) == 0
    rows_per_w = N // NW
    steps = rows_per_w // C
    assert steps % 2 == 0
    CL = C * L
    TOKS_W = rows_per_w * L
    mesh = plsc.VectorSubcoreMesh(core_axis_name="c", subcore_axis_name="s")

    @functools.partial(
        pl.kernel,
        out_type=jax.ShapeDtypeStruct((N, D), jnp.float32),
        mesh=mesh,
        scratch_types=[
            pltpu.VMEM((TOKS_W,), jnp.int32),
            pltpu.VMEM((CL, D), jnp.float32),
            pltpu.VMEM((CL, D), jnp.float32),
            pltpu.VMEM((C, D), jnp.float32),
            pltpu.VMEM((C, D), jnp.float32),
            pltpu.SemaphoreType.DMA,
            pltpu.SemaphoreType.DMA,
        ],
        compiler_params=pltpu.CompilerParams(use_tc_tiling_on_sc=False),
    )
    def gather_sum(idx_hbm, emb_hbm, out_hbm,
                   idxs, rows0, rows1, out0, out1, gsem0, gsem1):
        wid = lax.axis_index("s") * NC + lax.axis_index("c")
        row0_w = wid * rows_per_w

        pltpu.sync_copy(idx_hbm.at[pl.ds(row0_w * L, TOKS_W)], idxs)

        def gather(chunk, rows, sem):
            pltpu.async_copy(
                emb_hbm.at[idxs.at[pl.ds(chunk * CL, CL)]], rows, sem)

        def wait_gather(chunk, rows, sem):
            pltpu.make_async_copy(
                emb_hbm.at[idxs.at[pl.ds(chunk * CL, CL)]], rows, sem).wait()

        def reduce_chunk(chunk, rows, outv):
            def reduce_row(r, carry):
                base = r * L
                accs = [rows[base, pl.ds(k * LANES, LANES)]
                        for k in range(D // LANES)]
                for j in range(1, L):
                    accs = [a + rows[base + j, pl.ds(k * LANES, LANES)]
                            for k, a in enumerate(accs)]
                for k, a in enumerate(accs):
                    outv[r, pl.ds(k * LANES, LANES)] = a
                return carry

            lax.fori_loop(0, C, reduce_row, 0)
            pltpu.sync_copy(outv, out_hbm.at[pl.ds(row0_w + chunk * C, C)])

        gather(0, rows0, gsem0)

        def body(i, carry):
            c0 = 2 * i
            gather(c0 + 1, rows1, gsem1)
            wait_gather(c0, rows0, gsem0)
            reduce_chunk(c0, rows0, out0)
            n2 = jnp.minimum(c0 + 2, steps - 1)
            gather(n2, rows0, gsem0)
            wait_gather(c0 + 1, rows1, gsem1)
            reduce_chunk(c0 + 1, rows1, out1)
            return carry

        lax.fori_loop(0, steps // 2, body, 0)
        wait_gather(steps - 1, rows0, gsem0)

    return gather_sum


def _finish_body(tok_ref, sums_ref, w_ref, b_ref, out_ref):
    tok = tok_ref[...]
    cnt = jnp.sum((tok != 0).astype(jnp.float32), axis=1, keepdims=True)
    inv = 1.0 / jnp.maximum(cnt, 1.0)
    mean = sums_ref[...] * inv
    acc = lax.dot_general(mean, w_ref[...], (((1,), (1,)), ((), ())),
                          preferred_element_type=jnp.float32)
    out_ref[...] = jnp.maximum(acc + b_ref[...], 0.0)


def _finish(tok, sums, W, b, block_rows=2048):
    N, L = tok.shape
    assert N % block_rows == 0
    return pl.pallas_call(
        _finish_body,
        grid=(N // block_rows,),
        in_specs=[
            pl.BlockSpec((block_rows, L), lambda i: (i, 0)),
            pl.BlockSpec((block_rows, D), lambda i: (i, 0)),
            pl.BlockSpec((D, D), lambda i: (0, 0)),
            pl.BlockSpec((1, D), lambda i: (0, 0)),
        ],
        out_specs=pl.BlockSpec((block_rows, D), lambda i: (i, 0)),
        out_shape=jax.ShapeDtypeStruct((N, D), jnp.float32),
    )(tok, sums, W, b.reshape(1, D))


def kernel(sample, emb, W, b):
    L = sample.shape[-1]
    flat = sample.reshape(-1, L).astype(jnp.int32)
    N = flat.shape[0]
    toks = flat.reshape(-1)
    sums = _make_gather_sum(N, L, 8)(toks, emb)
    out = _finish(flat, sums, W, b)
    return out.reshape(sample.shape[:-1] + (D,))


# speedup vs baseline: 1.5740x; 1.3526x over previous
import functools

import jax
import jax.numpy as jnp
from jax import lax
from jax.experimental import pallas as pl
from jax.experimental.pallas import tpu as pltpu
from jax.experimental.pallas import tpu_sc as plsc

D = 64
LANES = 16


_RELAYOUT_BC = 4096
_RELAYOUT_HB = 123


def _relayout_body(a_ref, b_ref, out_ref):
    out_ref[:, 0:D] = a_ref[...].T
    out_ref[:, D:2 * D] = b_ref[...].T


def _relayout(embt):
    V = embt.shape[1]
    bc, hb = _RELAYOUT_BC, _RELAYOUT_HB
    h = bc * hb
    assert h < V <= 2 * h
    last = (V - 1) // bc
    return pl.pallas_call(
        _relayout_body,
        grid=(hb,),
        in_specs=[
            pl.BlockSpec((D, bc), lambda i: (0, i)),
            pl.BlockSpec((D, bc), lambda i: (0, jnp.minimum(i + hb, last))),
        ],
        out_specs=pl.BlockSpec((bc, 2 * D), lambda i: (i, 0)),
        out_shape=jax.ShapeDtypeStruct((h, 2 * D), jnp.float32),
    )(embt, embt)


@functools.lru_cache(maxsize=None)
def _make_gather_sum(N, L, C):
    NC, NS = 2, 16
    NW = NC * NS
    assert N % (NW * C) == 0
    rows_per_w = N // NW
    steps = rows_per_w // C
    assert steps % 2 == 0
    CL = C * L
    mesh = plsc.VectorSubcoreMesh(core_axis_name="c", subcore_axis_name="s")

    @functools.partial(
        pl.kernel,
        out_type=jax.ShapeDtypeStruct((N, D), jnp.float32),
        mesh=mesh,
        scratch_types=[
            pltpu.VMEM((CL,), jnp.int32),
            pltpu.VMEM((CL,), jnp.int32),
            pltpu.VMEM((CL + LANES,), jnp.int32),
            pltpu.VMEM((CL + LANES,), jnp.int32),
            pltpu.VMEM((CL, 2 * D), jnp.float32),
            pltpu.VMEM((CL, 2 * D), jnp.float32),
            pltpu.VMEM((C, D), jnp.float32),
            pltpu.VMEM((C, D), jnp.float32),
            pltpu.SemaphoreType.DMA,
            pltpu.SemaphoreType.DMA,
            pltpu.SemaphoreType.DMA,
            pltpu.SemaphoreType.DMA,
        ],
        compiler_params=pltpu.CompilerParams(use_tc_tiling_on_sc=True),
    )
    def gather_sum(idx2_hbm, offs_hbm, emb2_hbm, out_hbm,
                   idx0, idx1, off0, off1, rows0, rows1, out0, out1,
                   isem0, isem1, gsem0, gsem1):
        wid = lax.axis_index("s") * NC + lax.axis_index("c")
        row0_w = wid * rows_per_w
        tok0_w = row0_w * L

        def stage_idx(chunk, idxv, sem):
            pltpu.async_copy(
                idx2_hbm.at[pl.ds(tok0_w + chunk * CL, CL)], idxv, sem)

        def stage_off(chunk, offv, sem):
            pltpu.async_copy(
                offs_hbm.at[pl.ds(tok0_w + chunk * CL, CL)],
                offv.at[pl.ds(0, CL)], sem)

        def wait_stage(idxv, sem):
            pltpu.make_async_copy(
                idx2_hbm.at[pl.ds(tok0_w, CL)], idxv, sem).wait()

        def gather(idxv, rows, sem):
            pltpu.async_copy(emb2_hbm.at[idxv], rows, sem)

        def wait_gather(idxv, rows, sem):
            pltpu.make_async_copy(emb2_hbm.at[idxv], rows, sem).wait()

        def reduce_chunk(chunk, rows, offv, outv):
            def reduce_row(r, carry2):
                base = r * L

                def off_at(g):
                    return offv[pl.ds(g, LANES)][0]

                o0 = off_at(base)
                accs = [rows[base, pl.ds(o0 + k * LANES, LANES)]
                        for k in range(D // LANES)]
                for j in range(1, L):
                    oj = off_at(base + j)
                    accs = [a + rows[base + j, pl.ds(oj + k * LANES, LANES)]
                            for k, a in enumerate(accs)]
                for k, a in enumerate(accs):
                    outv[r, pl.ds(k * LANES, LANES)] = a
                return carry2

            lax.fori_loop(0, C, reduce_row, 0)
            pltpu.sync_copy(outv, out_hbm.at[pl.ds(row0_w + chunk * C, C)])

        pltpu.sync_copy(idx2_hbm.at[pl.ds(tok0_w, CL)], idx0)
        stage_off(0, off0, isem0)
        gather(idx0, rows0, gsem0)
        stage_idx(1, idx1, isem1)
        stage_off(1, off1, isem1)

        def body(i, carry):
            c0 = 2 * i
            n2 = jnp.minimum(c0 + 2, steps - 1)
            n3 = jnp.minimum(c0 + 3, steps - 1)
            wait_stage(idx1, isem1)
            wait_stage(idx1, isem1)
            gather(idx1, rows1, gsem1)
            wait_stage(idx0, isem0)
            wait_gather(idx0, rows0, gsem0)
            reduce_chunk(c0, rows0, off0, out0)
            stage_idx(n2, idx0, isem0)
            wait_stage(idx0, isem0)
            gather(idx0, rows0, gsem0)
            stage_off(n2, off0, isem0)
            wait_gather(idx1, rows1, gsem1)
            reduce_chunk(c0 + 1, rows1, off1, out1)
            stage_idx(n3, idx1, isem1)
            stage_off(n3, off1, isem1)
            return carry

        lax.fori_loop(0, steps // 2, body, 0)
        wait_stage(idx0, isem0)
        wait_gather(idx0, rows0, gsem0)
        wait_stage(idx1, isem1)
        wait_stage(idx1, isem1)

    return gather_sum


def _finish_body(tok_ref, sums_ref, w_ref, b_ref, out_ref):
    tok = tok_ref[...]
    cnt = jnp.sum((tok != 0).astype(jnp.float32), axis=1, keepdims=True)
    inv = 1.0 / jnp.maximum(cnt, 1.0)
    mean = sums_ref[...] * inv
    acc = lax.dot_general(mean, w_ref[...], (((1,), (1,)), ((), ())),
                          preferred_element_type=jnp.float32)
    out_ref[...] = jnp.maximum(acc + b_ref[...], 0.0)


def _finish(tok, sums, W, b, block_rows=2048):
    N, L = tok.shape
    assert N % block_rows == 0
    return pl.pallas_call(
        _finish_body,
        grid=(N // block_rows,),
        in_specs=[
            pl.BlockSpec((block_rows, L), lambda i: (i, 0)),
            pl.BlockSpec((block_rows, D), lambda i: (i, 0)),
            pl.BlockSpec((D, D), lambda i: (0, 0)),
            pl.BlockSpec((1, D), lambda i: (0, 0)),
        ],
        out_specs=pl.BlockSpec((block_rows, D), lambda i: (i, 0)),
        out_shape=jax.ShapeDtypeStruct((N, D), jnp.float32),
    )(tok, sums, W, b.reshape(1, D))


def kernel(sample, emb, W, b):
    L = sample.shape[-1]
    flat = sample.reshape(-1, L).astype(jnp.int32)
    N = flat.shape[0]
    toks = flat.reshape(-1)
    half = _RELAYOUT_BC * _RELAYOUT_HB
    in_hi = (toks >= half).astype(jnp.int32)
    idx2 = toks - in_hi * half
    offs = in_hi << 6
    emb2 = _relayout(emb.T)
    sums = _make_gather_sum(N, L, 8)(idx2, offs, emb2)
    out = _finish(flat, sums, W, b)
    return out.reshape(sample.shape[:-1] + (D,))


# speedup vs baseline: 1.9742x; 1.2543x over previous
import functools

import jax
import jax.numpy as jnp
from jax import lax
from jax.experimental import pallas as pl
from jax.experimental.pallas import tpu as pltpu
from jax.experimental.pallas import tpu_sc as plsc

D = 64
LANES = 16


_RELAYOUT_BC = 4096
_RELAYOUT_HB = 123


def _relayout_body(a_ref, b_ref, out_ref):
    out_ref[:, 0:D] = a_ref[...].T
    out_ref[:, D:2 * D] = b_ref[...].T


def _relayout(embt):
    V = embt.shape[1]
    bc, hb = _RELAYOUT_BC, _RELAYOUT_HB
    h = bc * hb
    assert h < V <= 2 * h
    last = (V - 1) // bc
    return pl.pallas_call(
        _relayout_body,
        grid=(hb,),
        in_specs=[
            pl.BlockSpec((D, bc), lambda i: (0, i)),
            pl.BlockSpec((D, bc), lambda i: (0, jnp.minimum(i + hb, last))),
        ],
        out_specs=pl.BlockSpec((bc, 2 * D), lambda i: (i, 0)),
        out_shape=jax.ShapeDtypeStruct((h, 2 * D), jnp.float32),
    )(embt, embt)


@functools.lru_cache(maxsize=None)
def _make_gather_sum(N, L, C):
    NC, NS = 2, 16
    NW = NC * NS
    assert N % (NW * C) == 0
    rows_per_w = N // NW
    steps = rows_per_w // C
    assert steps % 2 == 0
    CL = C * L
    TOKS_W = rows_per_w * L
    mesh = plsc.VectorSubcoreMesh(core_axis_name="c", subcore_axis_name="s")

    @functools.partial(
        pl.kernel,
        out_type=jax.ShapeDtypeStruct((N, D), jnp.float32),
        mesh=mesh,
        scratch_types=[
            pltpu.VMEM((TOKS_W,), jnp.int32),
            pltpu.VMEM((CL, D), jnp.float32),
            pltpu.VMEM((CL, D), jnp.float32),
            pltpu.VMEM((C, D), jnp.float32),
            pltpu.VMEM((C, D), jnp.float32),
            pltpu.SemaphoreType.DMA,
            pltpu.SemaphoreType.DMA,
        ],
        compiler_params=pltpu.CompilerParams(use_tc_tiling_on_sc=False),
    )
    def gather_sum(idx_hbm, emb_hbm, out_hbm,
                   idxs, rows0, rows1, out0, out1, gsem0, gsem1):
        wid = lax.axis_index("s") * NC + lax.axis_index("c")
        row0_w = wid * rows_per_w

        pltpu.sync_copy(idx_hbm.at[pl.ds(row0_w * L, TOKS_W)], idxs)

        def gather(chunk, rows, sem):
            pltpu.async_copy(
                emb_hbm.at[idxs.at[pl.ds(chunk * CL, CL)]], rows, sem)

        def wait_gather(chunk, rows, sem):
            pltpu.make_async_copy(
                emb_hbm.at[idxs.at[pl.ds(chunk * CL, CL)]], rows, sem).wait()

        def reduce_chunk(chunk, rows, outv):
            def reduce_row(r, carry):
                base = r * L
                accs = [rows[base, pl.ds(k * LANES, LANES)]
                        for k in range(D // LANES)]
                for j in range(1, L):
                    accs = [a + rows[base + j, pl.ds(k * LANES, LANES)]
                            for k, a in enumerate(accs)]
                for k, a in enumerate(accs):
                    outv[r, pl.ds(k * LANES, LANES)] = a
                return carry

            lax.fori_loop(0, C, reduce_row, 0)
            pltpu.sync_copy(outv, out_hbm.at[pl.ds(row0_w + chunk * C, C)])

        gather(0, rows0, gsem0)

        def body(i, carry):
            c0 = 2 * i
            gather(c0 + 1, rows1, gsem1)
            wait_gather(c0, rows0, gsem0)
            reduce_chunk(c0, rows0, out0)
            n2 = jnp.minimum(c0 + 2, steps - 1)
            gather(n2, rows0, gsem0)
            wait_gather(c0 + 1, rows1, gsem1)
            reduce_chunk(c0 + 1, rows1, out1)
            return carry

        lax.fori_loop(0, steps // 2, body, 0)
        wait_gather(steps - 1, rows0, gsem0)

    return gather_sum


def _finish_body(tok_ref, sums_ref, w_ref, b_ref, out_ref):
    tok = tok_ref[...]
    cnt = jnp.sum((tok != 0).astype(jnp.float32), axis=1, keepdims=True)
    inv = 1.0 / jnp.maximum(cnt, 1.0)
    mean = sums_ref[...] * inv
    acc = lax.dot_general(mean, w_ref[...], (((1,), (1,)), ((), ())),
                          preferred_element_type=jnp.float32)
    out_ref[...] = jnp.maximum(acc + b_ref[...], 0.0)


def _finish(tok, sums, W, b, block_rows=2048):
    N, L = tok.shape
    assert N % block_rows == 0
    return pl.pallas_call(
        _finish_body,
        grid=(N // block_rows,),
        in_specs=[
            pl.BlockSpec((block_rows, L), lambda i: (i, 0)),
            pl.BlockSpec((block_rows, D), lambda i: (i, 0)),
            pl.BlockSpec((D, D), lambda i: (0, 0)),
            pl.BlockSpec((1, D), lambda i: (0, 0)),
        ],
        out_specs=pl.BlockSpec((block_rows, D), lambda i: (i, 0)),
        out_shape=jax.ShapeDtypeStruct((N, D), jnp.float32),
    )(tok, sums, W, b.reshape(1, D))


def kernel(sample, emb, W, b):
    L = sample.shape[-1]
    flat = sample.reshape(-1, L).astype(jnp.int32)
    N = flat.shape[0]
    toks = flat.reshape(-1)
    half = _RELAYOUT_BC * _RELAYOUT_HB
    in_hi = (toks >= half).astype(jnp.int32)
    ridx = (toks - in_hi * half) * 2 + in_hi
    emb2 = _relayout(emb.T).reshape(2 * half, D)
    sums = _make_gather_sum(N, L, 8)(ridx, emb2)
    out = _finish(flat, sums, W, b)
    return out.reshape(sample.shape[:-1] + (D,))
